# Initial kernel scaffold; baseline (speedup 1.0000x reference)
import jax
import jax.numpy as jnp
from jax import lax
from jax.experimental import pallas as pl
from jax.experimental.pallas import tpu as pltpu
from jax.experimental.pallas import tpu_sc as plsc

N = 100000
E = 1600000
L = 16
NC = 2
NS = 16
NW = NC * NS
ROWS = ((E // 128 + NW - 1) // NW) * NW
RPW = ROWS // NW
KC = 23
OUTER = RPW // KC
E_PAD = ROWS * 128
N_ACC = 102400
GARB = N
ZROWS = (N_ACC // NS) // 128
RS = N // NS
R_TC = 2000
NBLK = N // R_TC
NG = 64


def _sc_mesh():
    return plsc.VectorSubcoreMesh(core_axis_name="c", subcore_axis_name="s",
                                  num_cores=NC, num_subcores=NS)


def _zero_acc(zbuf, acc, s):
    def zrow(i, carry):
        zbuf[i, :] = jnp.zeros((L,), jnp.float32)
        return carry
    lax.fori_loop(0, 128, zrow, 0)
    zbase = s * (N_ACC // NS)
    def zcp(j, carry):
        pltpu.sync_copy(zbuf, acc.at[pl.ds(zbase + j * 128, 128)])
        return carry
    lax.fori_loop(0, ZROWS, zcp, 0)


def _readout(acc, out0, out1, c, s):
    ro = s * RS
    @pl.when(c == 0)
    def _():
        pltpu.sync_copy(acc.at[pl.ds(ro, RS)], out0.at[pl.ds(ro, RS)])
    @pl.when(c == 1)
    def _():
        pltpu.sync_copy(acc.at[pl.ds(ro, RS)], out1.at[pl.ds(ro, RS)])


def _sc_hist(dst2d):
    def body(dst_ref, out0, out1, dstb, rows, zbuf, acc, sem):
        del sem
        c = lax.axis_index("c")
        s = lax.axis_index("s")
        def onerow(i, carry):
            rows[i, :] = jnp.ones((L,), jnp.float32)
            return carry
        lax.fori_loop(0, 128, onerow, 0)
        _zero_acc(zbuf, acc, s)
        plsc.subcore_barrier()
        rbase = (c * NS + s) * RPW
        def outer(o, carry):
            r0 = rbase + o * KC
            pltpu.sync_copy(dst_ref.at[pl.ds(r0, KC)], dstb)
            def inner(j, carry2):
                pltpu.sync_copy(rows, acc.at[dstb.at[j]], add=True)
                return carry2
            lax.fori_loop(0, KC, inner, 0)
            return carry
        lax.fori_loop(0, OUTER, outer, 0)
        plsc.subcore_barrier()
        _readout(acc, out0, out1, c, s)

    return pl.kernel(
        body,
        out_type=[jax.ShapeDtypeStruct((N, L), jnp.float32)] * 2,
        mesh=_sc_mesh(),
        scratch_types=[
            pltpu.VMEM((KC, 128), jnp.int32),
            pltpu.VMEM((128, L), jnp.float32),
            pltpu.VMEM((128, L), jnp.float32),
            pltpu.VMEM_SHARED((N_ACC, L), jnp.float32),
            pltpu.SemaphoreType.DMA,
        ],
    )(dst2d)


def _sc_agg(table, src2d, dst2d):
    def body(table_ref, src_ref, dst_ref, out0, out1, srcb, dstb, rows, zbuf,
             acc, sem):
        c = lax.axis_index("c")
        s = lax.axis_index("s")
        _zero_acc(zbuf, acc, s)
        plsc.subcore_barrier()
        rbase = (c * NS + s) * RPW
        def outer(o, carry):
            r0 = rbase + o * KC
            pltpu.sync_copy(src_ref.at[pl.ds(r0, KC)], srcb)
            pltpu.sync_copy(dst_ref.at[pl.ds(r0, KC)], dstb)
            def inner(j, carry2):
                pltpu.async_copy(table_ref.at[srcb.at[j]], rows, sem).wait()
                pltpu.sync_copy(rows, acc.at[dstb.at[j]], add=True)
                return carry2
            lax.fori_loop(0, KC, inner, 0)
            return carry
        lax.fori_loop(0, OUTER, outer, 0)
        plsc.subcore_barrier()
        _readout(acc, out0, out1, c, s)

    return pl.kernel(
        body,
        out_type=[jax.ShapeDtypeStruct((N, L), jnp.float32)] * 2,
        mesh=_sc_mesh(),
        scratch_types=[
            pltpu.VMEM((KC, 128), jnp.int32),
            pltpu.VMEM((KC, 128), jnp.int32),
            pltpu.VMEM((128, L), jnp.float32),
            pltpu.VMEM((128, L), jnp.float32),
            pltpu.VMEM_SHARED((N_ACC, L), jnp.float32),
            pltpu.SemaphoreType.DMA,
        ],
    )(table, src2d, dst2d)


def _relu(v):
    return jnp.maximum(v, 0.0)


def _dot(a, b):
    return lax.dot_general(a, b, (((1,), (0,)), ((), ())),
                           preferred_element_type=jnp.float32)


def _row_spec(d):
    return pl.BlockSpec((R_TC, d), lambda i: (i, 0))


def _full_spec(shape):
    return pl.BlockSpec(shape, lambda i: (0, 0))


def _tc_post_hist(h0, h1, x):
    def body(h0r, h1r, xr, t16r, miscr):
        deg = h0r[:, 0:1] + h1r[:, 0:1] + 1.0
        dinv = lax.rsqrt(deg)
        t = xr[...] * dinv
        t16r[...] = jnp.broadcast_to(t, (R_TC, L))
        miscr[...] = jnp.concatenate(
            [dinv, t, jnp.zeros((R_TC, 6), jnp.float32)], axis=1)

    return pl.pallas_call(
        body,
        grid=(NBLK,),
        in_specs=[_row_spec(L), _row_spec(L), _row_spec(1)],
        out_specs=[_row_spec(L), _row_spec(8)],
        out_shape=[jax.ShapeDtypeStruct((N, L), jnp.float32),
                   jax.ShapeDtypeStruct((N, 8), jnp.float32)],
    )(h0, h1, x)


def _tc_post_t(u0, u1, misc, static, W_up, b_up):
    def body(u0r, u1r, mr, str_, wr, br, lor, hir):
        u = u0r[:, 0:1] + u1r[:, 0:1]
        dinv = mr[:, 0:1]
        t = mr[:, 1:2]
        sarr = dinv * (u + t)
        x1 = sarr * wr[...] + br[...]
        lor[...] = x1[:, :L]
        hir[...] = jnp.concatenate([x1[:, L:24], str_[...]], axis=1)

    return pl.pallas_call(
        body,
        grid=(NBLK,),
        in_specs=[_row_spec(L), _row_spec(L), _row_spec(8), _row_spec(8),
                  _full_spec((1, 24)), _full_spec((1, 24))],
        out_specs=[_row_spec(L), _row_spec(L)],
        out_shape=[jax.ShapeDtypeStruct((N, L), jnp.float32),
                   jax.ShapeDtypeStruct((N, L), jnp.float32)],
    )(u0, u1, misc, static, W_up, b_up)


def _gin_mlps(hlor, hhir, a0r, a1r, b0r, b1r, epsr, gw1, gb1r, gw2, gb2r,
              gw3, gb3r, nw1, nb1r, nw2, nb2r, nw3, nb3r):
    h = jnp.concatenate([hlor[...], hhir[...]], axis=1)
    agg = jnp.concatenate([a0r[...] + a1r[...], b0r[...] + b1r[...]], axis=1)
    z = (1.0 + epsr[...]) * h + agg
    a = _relu(_dot(z, gw1[...]) + gb1r[...])
    a = _relu(_dot(a, gw2[...]) + gb2r[...])
    a = _relu(_dot(a, gw3[...]) + gb3r[...])
    st = hhir[:, 8:16]
    m = _relu(_dot(jnp.concatenate([a, st], axis=1), nw1[...]) + nb1r[...])
    m = _relu(_dot(m, nw2[...]) + nb2r[...])
    x2 = _relu(_dot(m, nw3[...]) + nb3r[...])
    return x2, st


_GIN_W_SPECS = [
    _full_spec((1, 1)),
    _full_spec((32, 32)), _full_spec((1, 32)),
    _full_spec((32, 24)), _full_spec((1, 24)),
    _full_spec((24, 24)), _full_spec((1, 24)),
    _full_spec((32, 24)), _full_spec((1, 24)),
    _full_spec((24, 24)), _full_spec((1, 24)),
    _full_spec((24, 24)), _full_spec((1, 24)),
]


def _tc_gin(hlo, hhi, alo0, alo1, ahi0, ahi1, *weights):
    def body(hlor, hhir, a0r, a1r, b0r, b1r, *rest):
        wrefs, (lor, hir) = rest[:-2], rest[-2:]
        x2, st = _gin_mlps(hlor, hhir, a0r, a1r, b0r, b1r, *wrefs)
        lor[...] = x2[:, :L]
        hir[...] = jnp.concatenate([x2[:, L:24], st], axis=1)

    return pl.pallas_call(
        body,
        grid=(NBLK,),
        in_specs=[_row_spec(L)] * 6 + _GIN_W_SPECS,
        out_specs=[_row_spec(L), _row_spec(L)],
        out_shape=[jax.ShapeDtypeStruct((N, L), jnp.float32),
                   jax.ShapeDtypeStruct((N, L), jnp.float32)],
    )(hlo, hhi, alo0, alo1, ahi0, ahi1, *weights)


def _tc_gin_final(hlo, hhi, alo0, alo1, ahi0, ahi1, bid, lW, lb, *weights):
    def body(hlor, hhir, a0r, a1r, b0r, b1r, bidr, lwr, lbr, *rest):
        wrefs, outr = rest[:-1], rest[-1]
        x2, _ = _gin_mlps(hlor, hhir, a0r, a1r, b0r, b1r, *wrefs)
        y = _dot(x2, lwr[...])
        eq = (bidr[...] == lax.broadcasted_iota(jnp.int32, (R_TC, NG), 1)
              ).astype(jnp.float32)
        contrib = lax.dot_general(eq, y, (((0,), (0,)), ((), ())),
                                  preferred_element_type=jnp.float32)
        i = pl.program_id(0)
        @pl.when(i == 0)
        def _():
            outr[...] = jnp.zeros((NG, 1), jnp.float32)
        outr[...] += contrib
        @pl.when(i == NBLK - 1)
        def _():
            outr[...] = _relu(outr[...] + lbr[...])

    return pl.pallas_call(
        body,
        grid=(NBLK,),
        in_specs=([_row_spec(L)] * 6
                  + [pl.BlockSpec((R_TC, 1), lambda i: (i, 0)),
                     _full_spec((24, 1)), _full_spec((1, 1))]
                  + _GIN_W_SPECS),
        out_specs=pl.BlockSpec((NG, 1), lambda i: (0, 0)),
        out_shape=jax.ShapeDtypeStruct((NG, 1), jnp.float32),
    )(hlo, hhi, alo0, alo1, ahi0, ahi1, bid, lW, lb, *weights)


def kernel(x, edge_index, inner_edges, feature_mtx_static, batch_ids,
           W_up, b_up, eps,
           gW1, gb1, gW2, gb2, gW3, gb3,
           nW1, nb1, nW2, nb2, nW3, nb3,
           lW, lb):
    pad = E_PAD - E

    def prep(ei):
        src = jnp.concatenate(
            [ei[0], jnp.zeros((pad,), jnp.int32)]).reshape(ROWS, 128)
        dst = jnp.concatenate(
            [ei[1], jnp.full((pad,), GARB, jnp.int32)]).reshape(ROWS, 128)
        return src, dst

    se, de = prep(edge_index)
    si, di = prep(inner_edges)

    weights = (eps.reshape(1, 1),
               gW1, gb1.reshape(1, -1), gW2, gb2.reshape(1, -1),
               gW3, gb3.reshape(1, -1), nW1, nb1.reshape(1, -1),
               nW2, nb2.reshape(1, -1), nW3, nb3.reshape(1, -1))

    h0, h1 = _sc_hist(de)
    t16, misc = _tc_post_hist(h0, h1, x)
    u0, u1 = _sc_agg(t16, se, de)
    hlo, hhi = _tc_post_t(u0, u1, misc, feature_mtx_static,
                          W_up, b_up.reshape(1, 24))

    alo0, alo1 = _sc_agg(hlo, si, di)
    ahi0, ahi1 = _sc_agg(hhi, si, di)
    hlo, hhi = _tc_gin(hlo, hhi, alo0, alo1, ahi0, ahi1, *weights)

    alo0, alo1 = _sc_agg(hlo, si, di)
    ahi0, ahi1 = _sc_agg(hhi, si, di)
    out = _tc_gin_final(hlo, hhi, alo0, alo1, ahi0, ahi1,
                        batch_ids.reshape(N, 1), lW, lb.reshape(1, 1),
                        *weights)
    return out


# speedup vs baseline: 13.7626x; 13.7626x over previous
import jax
import jax.numpy as jnp
from jax import lax
from jax.experimental import pallas as pl
from jax.experimental.pallas import tpu as pltpu
from jax.experimental.pallas import tpu_sc as plsc

N = 100000
E = 1600000
L = 16
NC = 2
NS = 16
NW = NC * NS
RPW = 392
ROWS = RPW * NW
KC = 56
OUTER = RPW // KC
E_PAD = ROWS * 128
N_ACC = 102400
GARB = N
ZROWS = (N_ACC // NS) // 128
N_OUT = 100096
RS = N_OUT // NS
R_TC = 2000
NBLK = N // R_TC
NG = 64


def _sc_mesh():
    return plsc.VectorSubcoreMesh(core_axis_name="c", subcore_axis_name="s",
                                  num_cores=NC, num_subcores=NS)


def _zero_acc(zbuf, acc, s):
    def zrow(i, carry):
        zbuf[i, :] = jnp.zeros((L,), jnp.float32)
        return carry
    lax.fori_loop(0, 128, zrow, 0)
    zbase = s * (N_ACC // NS)
    def zcp(j, carry):
        pltpu.sync_copy(zbuf, acc.at[pl.ds(zbase + j * 128, 128)])
        return carry
    lax.fori_loop(0, ZROWS, zcp, 0)


def _readout(acc, out0, out1, c, s):
    ro = s * RS
    @pl.when(c == 0)
    def _():
        pltpu.sync_copy(acc.at[pl.ds(ro, RS)], out0.at[pl.ds(ro, RS)])
    @pl.when(c == 1)
    def _():
        pltpu.sync_copy(acc.at[pl.ds(ro, RS)], out1.at[pl.ds(ro, RS)])


def _sc_hist(dst2d):
    def body(dst_ref, out0, out1, dstb, rows, zbuf, acc, sem):
        del sem
        c = lax.axis_index("c")
        s = lax.axis_index("s")
        def onerow(i, carry):
            rows[i, :] = jnp.ones((L,), jnp.float32)
            return carry
        lax.fori_loop(0, 128, onerow, 0)
        _zero_acc(zbuf, acc, s)
        plsc.subcore_barrier()
        rbase = (c * NS + s) * RPW
        def outer(o, carry):
            r0 = rbase + o * KC
            pltpu.sync_copy(dst_ref.at[pl.ds(r0, KC)], dstb)
            def inner(j, carry2):
                pltpu.sync_copy(rows, acc.at[dstb.at[j]], add=True)
                return carry2
            lax.fori_loop(0, KC, inner, 0)
            return carry
        lax.fori_loop(0, OUTER, outer, 0)
        plsc.subcore_barrier()
        _readout(acc, out0, out1, c, s)

    return pl.kernel(
        body,
        out_type=[jax.ShapeDtypeStruct((N_OUT, L), jnp.float32)] * 2,
        mesh=_sc_mesh(),
        compiler_params=pltpu.CompilerParams(use_tc_tiling_on_sc=False),
        scratch_types=[
            pltpu.VMEM((KC, 128), jnp.int32),
            pltpu.VMEM((128, L), jnp.float32),
            pltpu.VMEM((128, L), jnp.float32),
            pltpu.VMEM_SHARED((N_ACC, L), jnp.float32),
            pltpu.SemaphoreType.DMA,
        ],
    )(dst2d)


def _sc_agg(table, src2d, dst2d):
    def body(table_ref, src_ref, dst_ref, out0, out1, srcb, dstb, rows, zbuf,
             acc, sem):
        c = lax.axis_index("c")
        s = lax.axis_index("s")
        _zero_acc(zbuf, acc, s)
        plsc.subcore_barrier()
        rbase = (c * NS + s) * RPW
        def outer(o, carry):
            r0 = rbase + o * KC
            pltpu.sync_copy(src_ref.at[pl.ds(r0, KC)], srcb)
            pltpu.sync_copy(dst_ref.at[pl.ds(r0, KC)], dstb)
            def inner(j, carry2):
                pltpu.async_copy(table_ref.at[srcb.at[j]], rows, sem).wait()
                pltpu.sync_copy(rows, acc.at[dstb.at[j]], add=True)
                return carry2
            lax.fori_loop(0, KC, inner, 0)
            return carry
        lax.fori_loop(0, OUTER, outer, 0)
        plsc.subcore_barrier()
        _readout(acc, out0, out1, c, s)

    return pl.kernel(
        body,
        out_type=[jax.ShapeDtypeStruct((N_OUT, L), jnp.float32)] * 2,
        mesh=_sc_mesh(),
        compiler_params=pltpu.CompilerParams(use_tc_tiling_on_sc=False),
        scratch_types=[
            pltpu.VMEM((KC, 128), jnp.int32),
            pltpu.VMEM((KC, 128), jnp.int32),
            pltpu.VMEM((128, L), jnp.float32),
            pltpu.VMEM((128, L), jnp.float32),
            pltpu.VMEM_SHARED((N_ACC, L), jnp.float32),
            pltpu.SemaphoreType.DMA,
        ],
    )(table, src2d, dst2d)


def _relu(v):
    return jnp.maximum(v, 0.0)


def _dot(a, b):
    return lax.dot_general(a, b, (((1,), (0,)), ((), ())),
                           precision=lax.Precision.HIGHEST,
                           preferred_element_type=jnp.float32)


def _row_spec(d):
    return pl.BlockSpec((R_TC, d), lambda i: (i, 0))


def _full_spec(shape):
    return pl.BlockSpec(shape, lambda i: (0, 0))


def _tc_post_hist(h0, h1, x):
    def body(h0r, h1r, xr, t16r, miscr):
        deg = h0r[:, 0:1] + h1r[:, 0:1] + 1.0
        dinv = lax.rsqrt(deg)
        t = xr[...] * dinv
        t16r[...] = jnp.broadcast_to(t, (R_TC, L))
        miscr[...] = jnp.concatenate(
            [dinv, t, jnp.zeros((R_TC, 6), jnp.float32)], axis=1)

    return pl.pallas_call(
        body,
        grid=(NBLK,),
        in_specs=[_row_spec(L), _row_spec(L), _row_spec(1)],
        out_specs=[_row_spec(L), _row_spec(8)],
        out_shape=[jax.ShapeDtypeStruct((N, L), jnp.float32),
                   jax.ShapeDtypeStruct((N, 8), jnp.float32)],
    )(h0, h1, x)


def _tc_post_t(u0, u1, misc, static, W_up, b_up):
    def body(u0r, u1r, mr, str_, wr, br, lor, hir):
        u = u0r[:, 0:1] + u1r[:, 0:1]
        dinv = mr[:, 0:1]
        t = mr[:, 1:2]
        sarr = dinv * (u + t)
        x1 = sarr * wr[...] + br[...]
        lor[...] = x1[:, :L]
        hir[...] = jnp.concatenate([x1[:, L:24], str_[...]], axis=1)

    return pl.pallas_call(
        body,
        grid=(NBLK,),
        in_specs=[_row_spec(L), _row_spec(L), _row_spec(8), _row_spec(8),
                  _full_spec((1, 24)), _full_spec((1, 24))],
        out_specs=[_row_spec(L), _row_spec(L)],
        out_shape=[jax.ShapeDtypeStruct((N, L), jnp.float32),
                   jax.ShapeDtypeStruct((N, L), jnp.float32)],
    )(u0, u1, misc, static, W_up, b_up)


def _gin_mlps(hlor, hhir, a0r, a1r, b0r, b1r, epsr, gw1, gb1r, gw2, gb2r,
              gw3, gb3r, nw1, nb1r, nw2, nb2r, nw3, nb3r):
    h = jnp.concatenate([hlor[...], hhir[...]], axis=1)
    agg = jnp.concatenate([a0r[...] + a1r[...], b0r[...] + b1r[...]], axis=1)
    z = (1.0 + epsr[...]) * h + agg
    a = _relu(_dot(z, gw1[...]) + gb1r[...])
    a = _relu(_dot(a, gw2[...]) + gb2r[...])
    a = _relu(_dot(a, gw3[...]) + gb3r[...])
    st = hhir[:, 8:16]
    m = _relu(_dot(jnp.concatenate([a, st], axis=1), nw1[...]) + nb1r[...])
    m = _relu(_dot(m, nw2[...]) + nb2r[...])
    x2 = _relu(_dot(m, nw3[...]) + nb3r[...])
    return x2, st


_GIN_W_SPECS = [
    _full_spec((1, 1)),
    _full_spec((32, 32)), _full_spec((1, 32)),
    _full_spec((32, 24)), _full_spec((1, 24)),
    _full_spec((24, 24)), _full_spec((1, 24)),
    _full_spec((32, 24)), _full_spec((1, 24)),
    _full_spec((24, 24)), _full_spec((1, 24)),
    _full_spec((24, 24)), _full_spec((1, 24)),
]


def _tc_gin(hlo, hhi, alo0, alo1, ahi0, ahi1, *weights):
    def body(hlor, hhir, a0r, a1r, b0r, b1r, *rest):
        wrefs, (lor, hir) = rest[:-2], rest[-2:]
        x2, st = _gin_mlps(hlor, hhir, a0r, a1r, b0r, b1r, *wrefs)
        lor[...] = x2[:, :L]
        hir[...] = jnp.concatenate([x2[:, L:24], st], axis=1)

    return pl.pallas_call(
        body,
        grid=(NBLK,),
        in_specs=[_row_spec(L)] * 6 + _GIN_W_SPECS,
        out_specs=[_row_spec(L), _row_spec(L)],
        out_shape=[jax.ShapeDtypeStruct((N, L), jnp.float32),
                   jax.ShapeDtypeStruct((N, L), jnp.float32)],
    )(hlo, hhi, alo0, alo1, ahi0, ahi1, *weights)


def _tc_gin_final(hlo, hhi, alo0, alo1, ahi0, ahi1, bid, lW, lb, *weights):
    def body(hlor, hhir, a0r, a1r, b0r, b1r, bidr, lwr, lbr, *rest):
        wrefs, outr = rest[:-1], rest[-1]
        x2, _ = _gin_mlps(hlor, hhir, a0r, a1r, b0r, b1r, *wrefs)
        y = _dot(x2, lwr[...])
        eq = (bidr[...] == lax.broadcasted_iota(jnp.int32, (R_TC, NG), 1)
              ).astype(jnp.float32)
        contrib = lax.dot_general(eq, y, (((0,), (0,)), ((), ())),
                                  precision=lax.Precision.HIGHEST,
                                  preferred_element_type=jnp.float32)
        i = pl.program_id(0)
        @pl.when(i == 0)
        def _():
            outr[...] = jnp.zeros((NG, 1), jnp.float32)
        outr[...] += contrib
        @pl.when(i == NBLK - 1)
        def _():
            outr[...] = _relu(outr[...] + lbr[...])

    return pl.pallas_call(
        body,
        grid=(NBLK,),
        in_specs=([_row_spec(L)] * 6
                  + [pl.BlockSpec((R_TC, 1), lambda i: (i, 0)),
                     _full_spec((24, 1)), _full_spec((1, 1))]
                  + _GIN_W_SPECS),
        out_specs=pl.BlockSpec((NG, 1), lambda i: (0, 0)),
        out_shape=jax.ShapeDtypeStruct((NG, 1), jnp.float32),
    )(hlo, hhi, alo0, alo1, ahi0, ahi1, bid, lW, lb, *weights)


def kernel(x, edge_index, inner_edges, feature_mtx_static, batch_ids,
           W_up, b_up, eps,
           gW1, gb1, gW2, gb2, gW3, gb3,
           nW1, nb1, nW2, nb2, nW3, nb3,
           lW, lb):
    pad = E_PAD - E

    def prep(ei):
        src = jnp.concatenate(
            [ei[0], jnp.zeros((pad,), jnp.int32)]).reshape(ROWS, 128)
        dst = jnp.concatenate(
            [ei[1], jnp.full((pad,), GARB, jnp.int32)]).reshape(ROWS, 128)
        return src, dst

    se, de = prep(edge_index)
    si, di = prep(inner_edges)

    weights = (eps.reshape(1, 1),
               gW1, gb1.reshape(1, -1), gW2, gb2.reshape(1, -1),
               gW3, gb3.reshape(1, -1), nW1, nb1.reshape(1, -1),
               nW2, nb2.reshape(1, -1), nW3, nb3.reshape(1, -1))

    h0, h1 = _sc_hist(de)
    t16, misc = _tc_post_hist(h0, h1, x)
    u0, u1 = _sc_agg(t16, se, de)
    hlo, hhi = _tc_post_t(u0, u1, misc, feature_mtx_static,
                          W_up, b_up.reshape(1, 24))

    alo0, alo1 = _sc_agg(hlo, si, di)
    ahi0, ahi1 = _sc_agg(hhi, si, di)
    hlo, hhi = _tc_gin(hlo, hhi, alo0, alo1, ahi0, ahi1, *weights)

    alo0, alo1 = _sc_agg(hlo, si, di)
    ahi0, ahi1 = _sc_agg(hhi, si, di)
    out = _tc_gin_final(hlo, hhi, alo0, alo1, ahi0, ahi1,
                        batch_ids.reshape(N, 1), lW, lb.reshape(1, 1),
                        *weights)
    return out


# speedup vs baseline: 25.4849x; 1.8518x over previous
import jax
import jax.numpy as jnp
from jax import lax
from jax.experimental import pallas as pl
from jax.experimental.pallas import tpu as pltpu
from jax.experimental.pallas import tpu_sc as plsc

N = 100000
E = 1600000
L = 16
NC = 2
NS = 16
NW = NC * NS
RPW = 392
ROWS = RPW * NW
CW = 128
ROWS_W = ROWS * 128 // CW
RPW_W = RPW * 128 // CW
KC = 56
OUTER = RPW_W // KC
SPC = KC
E_PAD = ROWS * 128
N_ACC = 100096
GARB = N
ZROWS = (N_ACC // NS) // 128
N_OUT = 100096
RS = N_OUT // NS
R_TC = 2000
NBLK = N // R_TC
NG = 64


def _sc_mesh():
    return plsc.VectorSubcoreMesh(core_axis_name="c", subcore_axis_name="s",
                                  num_cores=NC, num_subcores=NS)


def _zero_acc(zbuf, acc, s):
    def zrow(i, carry):
        zbuf[i, :] = jnp.zeros((L,), jnp.float32)
        return carry
    lax.fori_loop(0, 128, zrow, 0)
    zbase = s * (N_ACC // NS)
    def zcp(j, carry):
        pltpu.sync_copy(zbuf, acc.at[pl.ds(zbase + j * 128, 128)])
        return carry
    lax.fori_loop(0, (N_ACC // NS) // 128, zcp, 0)
    rem = (N_ACC // NS) % 128
    if rem:
        pltpu.sync_copy(
            zbuf.at[pl.ds(0, rem)],
            acc.at[pl.ds(zbase + (N_ACC // NS) // 128 * 128, rem)])


def _readout(acc, out0, out1, c, s):
    ro = s * RS
    @pl.when(c == 0)
    def _():
        pltpu.sync_copy(acc.at[pl.ds(ro, RS)], out0.at[pl.ds(ro, RS)])
    @pl.when(c == 1)
    def _():
        pltpu.sync_copy(acc.at[pl.ds(ro, RS)], out1.at[pl.ds(ro, RS)])


def _sc_hist(dst2d):
    def body(dst_ref, out0, out1, dstb, ones3, zbuf, acc, sem_s):
        c = lax.axis_index("c")
        s = lax.axis_index("s")
        _zero_acc(zbuf, acc, s)
        def onerow(i, carry):
            ones3[i, :] = jnp.ones((L,), jnp.float32)
            return carry
        lax.fori_loop(0, CW, onerow, 0)
        plsc.subcore_barrier()
        rbase = (c * NS + s) * RPW_W
        def outer(o, carry):
            r0 = rbase + o * KC
            pltpu.sync_copy(dst_ref.at[pl.ds(r0, KC)], dstb)
            descs = []
            for u in range(SPC):
                if u >= 8:
                    descs[u - 8].wait()
                descs.append(pltpu.async_copy(
                    ones3, acc.at[dstb.at[u]], sem_s, add=True))
            for u in range(max(0, SPC - 8), SPC):
                descs[u].wait()
            return carry
        lax.fori_loop(0, OUTER, outer, 0)
        plsc.subcore_barrier()
        _readout(acc, out0, out1, c, s)

    return pl.kernel(
        body,
        out_type=[jax.ShapeDtypeStruct((N_OUT, L), jnp.float32)] * 2,
        mesh=_sc_mesh(),
        compiler_params=pltpu.CompilerParams(use_tc_tiling_on_sc=False),
        scratch_types=[
            pltpu.VMEM((KC, CW), jnp.int32),
            pltpu.VMEM((CW, L), jnp.float32),
            pltpu.VMEM((128, L), jnp.float32),
            pltpu.VMEM_SHARED((N_ACC, L), jnp.float32),
            pltpu.SemaphoreType.DMA,
        ],
    )(dst2d)


def _sc_agg(table, src2d, dst2d):
    def body(table_ref, src_ref, dst_ref, out0, out1, srcb, dstb, rowsA,
             rowsB, rowsC, rowsD, zbuf, acc,
             g0, g1, g2, g3, s0, s1, s2, s3):
        gsems = (g0, g1, g2, g3)
        ssems = (s0, s1, s2, s3)
        c = lax.axis_index("c")
        s = lax.axis_index("s")
        _zero_acc(zbuf, acc, s)
        plsc.subcore_barrier()
        rbase = (c * NS + s) * RPW_W
        ring = (rowsA, rowsB, rowsC, rowsD)
        NB = 4
        def outer(o, carry):
            r0 = rbase + o * KC
            pltpu.sync_copy(src_ref.at[pl.ds(r0, KC)], srcb)
            pltpu.sync_copy(dst_ref.at[pl.ds(r0, KC)], dstb)
            G = NB // 2
            g = [None] * SPC
            sc = [None] * SPC
            for w in range(min(G, SPC)):
                g[w] = pltpu.async_copy(
                    table_ref.at[srcb.at[w]], ring[w % NB], gsems[w % NB])
            for u in range(SPC):
                g[u].wait()
                if u >= G:
                    sc[u - G].wait()
                sc[u] = pltpu.async_copy(
                    ring[u % NB], acc.at[dstb.at[u]], ssems[u % NB],
                    add=True)
                nxt = u + G
                if nxt < SPC:
                    g[nxt] = pltpu.async_copy(
                        table_ref.at[srcb.at[nxt]], ring[nxt % NB],
                        gsems[nxt % NB])
            for u in range(max(0, SPC - G), SPC):
                sc[u].wait()
            return carry
        lax.fori_loop(0, OUTER, outer, 0)
        plsc.subcore_barrier()
        _readout(acc, out0, out1, c, s)

    return pl.kernel(
        body,
        out_type=[jax.ShapeDtypeStruct((N_OUT, L), jnp.float32)] * 2,
        mesh=_sc_mesh(),
        compiler_params=pltpu.CompilerParams(use_tc_tiling_on_sc=False),
        scratch_types=[
            pltpu.VMEM((KC, CW), jnp.int32),
            pltpu.VMEM((KC, CW), jnp.int32),
            pltpu.VMEM((CW, L), jnp.float32),
            pltpu.VMEM((CW, L), jnp.float32),
            pltpu.VMEM((CW, L), jnp.float32),
            pltpu.VMEM((CW, L), jnp.float32),
            pltpu.VMEM((128, L), jnp.float32),
            pltpu.VMEM_SHARED((N_ACC, L), jnp.float32),
        ] + [pltpu.SemaphoreType.DMA] * 8,
    )(table, src2d, dst2d)


def _relu(v):
    return jnp.maximum(v, 0.0)


def _dot(a, b):
    return lax.dot_general(a, b, (((1,), (0,)), ((), ())),
                           preferred_element_type=jnp.float32)


def _row_spec(d):
    return pl.BlockSpec((R_TC, d), lambda i: (i, 0))


def _full_spec(shape):
    return pl.BlockSpec(shape, lambda i: (0, 0))


def _tc_post_hist(h0, h1, x):
    def body(h0r, h1r, xr, t16r, miscr):
        deg = h0r[:, 0:1] + h1r[:, 0:1] + 1.0
        dinv = lax.rsqrt(deg)
        t = xr[...] * dinv
        t16r[...] = jnp.broadcast_to(t, (R_TC, L))
        miscr[...] = jnp.concatenate(
            [dinv, t, jnp.zeros((R_TC, 6), jnp.float32)], axis=1)

    return pl.pallas_call(
        body,
        grid=(NBLK,),
        in_specs=[_row_spec(L), _row_spec(L), _row_spec(1)],
        out_specs=[_row_spec(L), _row_spec(8)],
        out_shape=[jax.ShapeDtypeStruct((N, L), jnp.float32),
                   jax.ShapeDtypeStruct((N, 8), jnp.float32)],
    )(h0, h1, x)


def _tc_post_t(u0, u1, misc, static, W_up, b_up):
    def body(u0r, u1r, mr, str_, wr, br, lor, hir):
        u = u0r[:, 0:1] + u1r[:, 0:1]
        dinv = mr[:, 0:1]
        t = mr[:, 1:2]
        sarr = dinv * (u + t)
        x1 = sarr * wr[...] + br[...]
        lor[...] = x1[:, :L]
        hir[...] = jnp.concatenate([x1[:, L:24], str_[...]], axis=1)

    return pl.pallas_call(
        body,
        grid=(NBLK,),
        in_specs=[_row_spec(L), _row_spec(L), _row_spec(8), _row_spec(8),
                  _full_spec((1, 24)), _full_spec((1, 24))],
        out_specs=[_row_spec(L), _row_spec(L)],
        out_shape=[jax.ShapeDtypeStruct((N, L), jnp.float32),
                   jax.ShapeDtypeStruct((N, L), jnp.float32)],
    )(u0, u1, misc, static, W_up, b_up)


def _gin_mlps(hlor, hhir, a0r, a1r, b0r, b1r, epsr, gw1, gb1r, gw2, gb2r,
              gw3, gb3r, nw1, nb1r, nw2, nb2r, nw3, nb3r):
    h = jnp.concatenate([hlor[...], hhir[...]], axis=1)
    agg = jnp.concatenate([a0r[...] + a1r[...], b0r[...] + b1r[...]], axis=1)
    z = (1.0 + epsr[...]) * h + agg
    a = _relu(_dot(z, gw1[...]) + gb1r[...])
    a = _relu(_dot(a, gw2[...]) + gb2r[...])
    a = _relu(_dot(a, gw3[...]) + gb3r[...])
    st = hhir[:, 8:16]
    m = _relu(_dot(jnp.concatenate([a, st], axis=1), nw1[...]) + nb1r[...])
    m = _relu(_dot(m, nw2[...]) + nb2r[...])
    x2 = _relu(_dot(m, nw3[...]) + nb3r[...])
    return x2, st


_GIN_W_SPECS = [
    _full_spec((1, 1)),
    _full_spec((32, 32)), _full_spec((1, 32)),
    _full_spec((32, 24)), _full_spec((1, 24)),
    _full_spec((24, 24)), _full_spec((1, 24)),
    _full_spec((32, 24)), _full_spec((1, 24)),
    _full_spec((24, 24)), _full_spec((1, 24)),
    _full_spec((24, 24)), _full_spec((1, 24)),
]


def _tc_gin(hlo, hhi, alo0, alo1, ahi0, ahi1, *weights):
    def body(hlor, hhir, a0r, a1r, b0r, b1r, *rest):
        wrefs, (lor, hir) = rest[:-2], rest[-2:]
        x2, st = _gin_mlps(hlor, hhir, a0r, a1r, b0r, b1r, *wrefs)
        lor[...] = x2[:, :L]
        hir[...] = jnp.concatenate([x2[:, L:24], st], axis=1)

    return pl.pallas_call(
        body,
        grid=(NBLK,),
        in_specs=[_row_spec(L)] * 6 + _GIN_W_SPECS,
        out_specs=[_row_spec(L), _row_spec(L)],
        out_shape=[jax.ShapeDtypeStruct((N, L), jnp.float32),
                   jax.ShapeDtypeStruct((N, L), jnp.float32)],
    )(hlo, hhi, alo0, alo1, ahi0, ahi1, *weights)


def _tc_gin_final(hlo, hhi, alo0, alo1, ahi0, ahi1, bid, lW, lb, *weights):
    def body(hlor, hhir, a0r, a1r, b0r, b1r, bidr, lwr, lbr, *rest):
        wrefs, outr = rest[:-1], rest[-1]
        x2, _ = _gin_mlps(hlor, hhir, a0r, a1r, b0r, b1r, *wrefs)
        y = _dot(x2, lwr[...])
        eq = (bidr[...] == lax.broadcasted_iota(jnp.int32, (R_TC, NG), 1)
              ).astype(jnp.float32)
        contrib = lax.dot_general(eq, y, (((0,), (0,)), ((), ())),
                                  preferred_element_type=jnp.float32)
        i = pl.program_id(0)
        @pl.when(i == 0)
        def _():
            outr[...] = jnp.zeros((NG, 1), jnp.float32)
        outr[...] += contrib
        @pl.when(i == NBLK - 1)
        def _():
            outr[...] = _relu(outr[...] + lbr[...])

    return pl.pallas_call(
        body,
        grid=(NBLK,),
        in_specs=([_row_spec(L)] * 6
                  + [pl.BlockSpec((R_TC, 1), lambda i: (i, 0)),
                     _full_spec((24, 1)), _full_spec((1, 1))]
                  + _GIN_W_SPECS),
        out_specs=pl.BlockSpec((NG, 1), lambda i: (0, 0)),
        out_shape=jax.ShapeDtypeStruct((NG, 1), jnp.float32),
    )(hlo, hhi, alo0, alo1, ahi0, ahi1, bid, lW, lb, *weights)


def kernel(x, edge_index, inner_edges, feature_mtx_static, batch_ids,
           W_up, b_up, eps,
           gW1, gb1, gW2, gb2, gW3, gb3,
           nW1, nb1, nW2, nb2, nW3, nb3,
           lW, lb):
    pad = E_PAD - E

    def prep(ei):
        src = jnp.concatenate(
            [ei[0], jnp.zeros((pad,), jnp.int32)]).reshape(ROWS_W, CW)
        dst = jnp.concatenate(
            [ei[1], jnp.full((pad,), GARB, jnp.int32)]).reshape(ROWS_W, CW)
        return src, dst

    se, de = prep(edge_index)
    si, di = prep(inner_edges)

    weights = (eps.reshape(1, 1),
               gW1, gb1.reshape(1, -1), gW2, gb2.reshape(1, -1),
               gW3, gb3.reshape(1, -1), nW1, nb1.reshape(1, -1),
               nW2, nb2.reshape(1, -1), nW3, nb3.reshape(1, -1))

    h0, h1 = _sc_hist(de)
    t16, misc = _tc_post_hist(h0, h1, x)
    u0, u1 = _sc_agg(t16, se, de)
    hlo, hhi = _tc_post_t(u0, u1, misc, feature_mtx_static,
                          W_up, b_up.reshape(1, 24))

    alo0, alo1 = _sc_agg(hlo, si, di)
    ahi0, ahi1 = _sc_agg(hhi, si, di)
    hlo, hhi = _tc_gin(hlo, hhi, alo0, alo1, ahi0, ahi1, *weights)

    alo0, alo1 = _sc_agg(hlo, si, di)
    ahi0, ahi1 = _sc_agg(hhi, si, di)
    out = _tc_gin_final(hlo, hhi, alo0, alo1, ahi0, ahi1,
                        batch_ids.reshape(N, 1), lW, lb.reshape(1, 1),
                        *weights)
    return out
